# Initial kernel scaffold; baseline (speedup 1.0000x reference)
import jax
import jax.numpy as jnp
from jax.experimental import pallas as pl


def kernel(x, pos, emb):
    raise NotImplementedError("write your pallas kernel here")


# speedup vs baseline: 1.2482x; 1.2482x over previous
import functools

import jax
import jax.numpy as jnp
from jax import lax
from jax.experimental import pallas as pl
from jax.experimental.pallas import tpu as pltpu
from jax.experimental.pallas import tpu_sc as plsc

_SEQ = 8192
_DIM = 1024
_LANES = 16
_NC = 2
_NS = 16
_NW = _NC * _NS
_ROWS_PER_W = _SEQ // _NW
_CHUNK = 64
_N_CHUNKS = _ROWS_PER_W // _CHUNK
_VECS_PER_CHUNK = _CHUNK * _DIM // _LANES
_SCALE = _DIM ** -0.5


def _sc_embed(emb, idx):
    mesh = plsc.VectorSubcoreMesh(
        core_axis_name="c", subcore_axis_name="s",
        num_cores=_NC, num_subcores=_NS)

    @functools.partial(
        pl.kernel,
        out_type=jax.ShapeDtypeStruct((_SEQ, _DIM), jnp.float32),
        mesh=mesh,
        scratch_types=[
            pltpu.VMEM((_ROWS_PER_W,), jnp.int32),
            pltpu.VMEM((_CHUNK, _DIM), jnp.float32),
            pltpu.SemaphoreType.DMA,
        ],
    )
    def body(emb_hbm, idx_hbm, out_hbm, idx_v, rows_v, sem):
        wid = lax.axis_index("s") * _NC + lax.axis_index("c")
        base = wid * _ROWS_PER_W
        pltpu.sync_copy(idx_hbm.at[pl.ds(base, _ROWS_PER_W)], idx_v)

        for ci in range(_N_CHUNKS):
            row0 = ci * _CHUNK
            pltpu.async_copy(
                emb_hbm.at[idx_v.at[pl.ds(row0, _CHUNK)]], rows_v, sem).wait()

            @plsc.parallel_loop(0, _VECS_PER_CHUNK, unroll=8)
            def _scale_vec(k):
                r = k // (_DIM // _LANES)
                j = k % (_DIM // _LANES)
                sl = pl.ds(j * _LANES, _LANES)
                rows_v[r, sl] = rows_v[r, sl] * _SCALE

            pltpu.sync_copy(rows_v, out_hbm.at[pl.ds(base + row0, _CHUNK)])

    return body(emb, idx)


def kernel(x, pos, emb):
    del x
    return _sc_embed(emb, pos.astype(jnp.int32))


# speedup vs baseline: 1.4983x; 1.2003x over previous
import functools

import jax
import jax.numpy as jnp
from jax import lax
from jax.experimental import pallas as pl
from jax.experimental.pallas import tpu as pltpu
from jax.experimental.pallas import tpu_sc as plsc

_SEQ = 8192
_DIM = 1024
_LANES = 16
_NC = 2
_NS = 16
_NW = _NC * _NS
_ROWS_PER_W = _SEQ // _NW
_CHUNK = 16
_N_CHUNKS = _ROWS_PER_W // _CHUNK
_NBUF = 4
_VECS_PER_CHUNK = _CHUNK * _DIM // _LANES
_SCALE = _DIM ** -0.5


def _sc_embed(emb, idx):
    mesh = plsc.VectorSubcoreMesh(
        core_axis_name="c", subcore_axis_name="s",
        num_cores=_NC, num_subcores=_NS)

    @functools.partial(
        pl.kernel,
        out_type=jax.ShapeDtypeStruct((_SEQ, _DIM), jnp.float32),
        mesh=mesh,
        scratch_types=[
            pltpu.VMEM((_ROWS_PER_W,), jnp.int32),
            [pltpu.VMEM((_CHUNK, _DIM), jnp.float32)] * _NBUF,
            [pltpu.SemaphoreType.DMA] * _NBUF,
            [pltpu.SemaphoreType.DMA] * _NBUF,
        ],
    )
    def body(emb_hbm, idx_hbm, out_hbm, idx_v, bufs, gsems, ssems):
        wid = lax.axis_index("s") * _NC + lax.axis_index("c")
        base = wid * _ROWS_PER_W
        pltpu.sync_copy(idx_hbm.at[pl.ds(base, _ROWS_PER_W)], idx_v)

        def gather(ci):
            b = ci % _NBUF
            return pltpu.async_copy(
                emb_hbm.at[idx_v.at[pl.ds(ci * _CHUNK, _CHUNK)]],
                bufs[b], gsems[b])

        def scatter(ci):
            b = ci % _NBUF
            return pltpu.async_copy(
                bufs[b], out_hbm.at[pl.ds(base + ci * _CHUNK, _CHUNK)],
                ssems[b])

        gd = {}
        sd = {}
        for ci in range(_NBUF - 1):
            gd[ci] = gather(ci)
        for ci in range(_N_CHUNKS):
            nxt = ci + _NBUF - 1
            if nxt < _N_CHUNKS:
                if ci > 0:
                    sd[ci - 1].wait()
                gd[nxt] = gather(nxt)
            gd[ci].wait()
            buf = bufs[ci % _NBUF]

            @plsc.parallel_loop(0, _VECS_PER_CHUNK, unroll=8)
            def _scale_vec(k):
                r = k // (_DIM // _LANES)
                j = k % (_DIM // _LANES)
                sl = pl.ds(j * _LANES, _LANES)
                buf[r, sl] = buf[r, sl] * _SCALE

            sd[ci] = scatter(ci)
        for ci in range(_N_CHUNKS - _NBUF, _N_CHUNKS):
            sd[ci].wait()

    return body(emb, idx)


def kernel(x, pos, emb):
    del x
    return _sc_embed(emb, pos.astype(jnp.int32))
